# baseline (speedup 1.0000x reference)
import functools

import jax
import jax.numpy as jnp
from jax import lax
from jax.experimental import pallas as pl
from jax.experimental.pallas import tpu as pltpu
from jax.experimental.pallas import tpu_sc as plsc

_NUM_CORES = 2
_NUM_SUBCORES = 16
_NW = _NUM_CORES * _NUM_SUBCORES
_RING = 16


def _gather_tilecols(D: int, B: int, b_per_w: int):
    mesh = plsc.VectorSubcoreMesh(core_axis_name="c", subcore_axis_name="s")

    @functools.partial(
        pl.kernel,
        mesh=mesh,
        out_type=jax.ShapeDtypeStruct((D, B), jnp.float32),
        scratch_types=[
            pltpu.VMEM((b_per_w,), jnp.int32),
            pltpu.VMEM((_RING, D, 128), jnp.float32),
            pltpu.VMEM((D, b_per_w), jnp.float32),
            [pltpu.SemaphoreType.DMA] * _RING,
        ],
        compiler_params=pltpu.CompilerParams(
            use_tc_tiling_on_sc=True, needs_layout_passes=False),
    )
    def body(idx_hbm, wt_hbm, out_hbm, idx_v, ring_v, out_v, sems):
        wid = lax.axis_index("s") * _NUM_CORES + lax.axis_index("c")
        base = wid * b_per_w
        pltpu.sync_copy(idx_hbm.at[pl.ds(base, b_per_w)], idx_v)

        rows_lo = lax.iota(jnp.int32, 16)
        rows_hi = rows_lo + 16

        def sidx(g):
            start = pl.multiple_of((g >> 4) << 4, 16)
            chunk = idx_v[pl.ds(start, 16)]
            return jnp.max(jnp.where(rows_lo == (g & 15), chunk, -1))

        def fire(g, slot):
            v = sidx(g)
            col = pl.multiple_of((v >> 7) << 7, 128)
            pltpu.async_copy(
                wt_hbm.at[:, pl.ds(col, 128)], ring_v.at[slot], sems[slot])

        for s in range(_RING):
            fire(s, s)

        def step(b, carry):
            for s in range(_RING):
                g = b * _RING + s
                pltpu.make_async_copy(
                    wt_hbm.at[:, pl.ds(0, 128)], ring_v.at[s], sems[s]
                ).wait()
                c = jnp.full((16,), sidx(g) & 127, jnp.int32)
                gcol = jnp.full((16,), g, jnp.int32)
                lo = plsc.load_gather(ring_v.at[s], [rows_lo, c])
                hi = plsc.load_gather(ring_v.at[s], [rows_hi, c])
                plsc.store_scatter(out_v, [rows_lo, gcol], lo)
                plsc.store_scatter(out_v, [rows_hi, gcol], hi)

                @pl.when(g + _RING < b_per_w)
                def _():
                    fire(g + _RING, s)

            return carry

        lax.fori_loop(0, b_per_w // _RING, step, 0)
        pltpu.sync_copy(out_v, out_hbm.at[:, pl.ds(base, b_per_w)])

    return body


def kernel(category, embedding_weight):
    (B,) = category.shape
    V, D = embedding_weight.shape
    b_per_w = B // _NW
    out_t = _gather_tilecols(D, B, b_per_w)(category, embedding_weight.T)
    return out_t.T


# speedup vs baseline: 1.0227x; 1.0227x over previous
import functools

import jax
import jax.numpy as jnp
from jax import lax
from jax.experimental import pallas as pl
from jax.experimental.pallas import tpu as pltpu
from jax.experimental.pallas import tpu_sc as plsc

_NUM_CORES = 2
_NUM_SUBCORES = 16
_NW = _NUM_CORES * _NUM_SUBCORES
_RING = 8


def _gather_tilecols(D: int, B: int, b_per_w: int):
    mesh = plsc.VectorSubcoreMesh(core_axis_name="c", subcore_axis_name="s")

    @functools.partial(
        pl.kernel,
        mesh=mesh,
        out_type=jax.ShapeDtypeStruct((D, B), jnp.float32),
        scratch_types=[
            pltpu.VMEM((b_per_w,), jnp.int32),
            pltpu.VMEM((_RING, D, 128), jnp.float32),
            pltpu.VMEM((D, b_per_w), jnp.float32),
            [pltpu.SemaphoreType.DMA] * _RING,
        ],
        compiler_params=pltpu.CompilerParams(
            use_tc_tiling_on_sc=True, needs_layout_passes=False),
    )
    def body(idx_hbm, wt_hbm, out_hbm, idx_v, ring_v, out_v, sems):
        wid = lax.axis_index("s") * _NUM_CORES + lax.axis_index("c")
        base = wid * b_per_w
        pltpu.sync_copy(idx_hbm.at[pl.ds(base, b_per_w)], idx_v)

        rows_lo = lax.iota(jnp.int32, 16)
        rows_hi = rows_lo + 16

        def sidx(g):
            start = pl.multiple_of((g >> 4) << 4, 16)
            chunk = idx_v[pl.ds(start, 16)]
            return jnp.max(jnp.where(rows_lo == (g & 15), chunk, -1))

        def fire(g, slot):
            v = sidx(g)
            col = pl.multiple_of((v >> 7) << 7, 128)
            for t in range(D // 8):
                pltpu.async_copy(
                    wt_hbm.at[pl.ds(8 * t, 8), pl.ds(col, 128)],
                    ring_v.at[slot, pl.ds(8 * t, 8)],
                    sems[slot])

        for s in range(_RING):
            fire(s, s)

        def step(b, carry):
            for s in range(_RING):
                g = b * _RING + s
                pltpu.make_async_copy(
                    wt_hbm.at[:, pl.ds(0, 128)], ring_v.at[s], sems[s]
                ).wait()
                c = jnp.full((16,), sidx(g) & 127, jnp.int32)
                gcol = jnp.full((16,), g, jnp.int32)
                lo = plsc.load_gather(ring_v.at[s], [rows_lo, c])
                hi = plsc.load_gather(ring_v.at[s], [rows_hi, c])
                plsc.store_scatter(out_v, [rows_lo, gcol], lo)
                plsc.store_scatter(out_v, [rows_hi, gcol], hi)

                @pl.when(g + _RING < b_per_w)
                def _():
                    fire(g + _RING, s)

            return carry

        lax.fori_loop(0, b_per_w // _RING, step, 0)
        pltpu.sync_copy(out_v, out_hbm.at[:, pl.ds(base, b_per_w)])

    return body


def kernel(category, embedding_weight):
    (B,) = category.shape
    V, D = embedding_weight.shape
    b_per_w = B // _NW
    out_t = _gather_tilecols(D, B, b_per_w)(category, embedding_weight.T)
    return out_t.T
